# baseline (speedup 1.0000x reference)
import functools

import jax
import jax.numpy as jnp
from jax import lax
from jax.experimental import pallas as pl
from jax.experimental.pallas import tpu as pltpu
from jax.experimental.pallas import tpu_sc as plsc

_N = 10000
_H = 128
_G = 64
_NP = 10240
_E = 320000
_EP = 331776
_NW = 32
_HC = _H // 2
_KC = 128
_NCH = _EP // 16 // _KC
_RPT = _NP // 16
_BPW = _NP // _NW
_KE = 64
_NE = _BPW // _KE

@functools.cache
def _sc_embed_kernel():
    mesh = plsc.VectorSubcoreMesh(core_axis_name="c", subcore_axis_name="s")

    @functools.partial(
        pl.kernel, mesh=mesh,
        out_type=jax.ShapeDtypeStruct((_NP, _H), jnp.float32),
        scratch_types=[
            pltpu.VMEM((_NE, _KE), jnp.int32),
            pltpu.VMEM((_KE, _H), jnp.float32),
            pltpu.SemaphoreType.DMA,
        ],
        compiler_params=pltpu.CompilerParams(needs_layout_passes=False),
    )
    def _sc_embed(tab, idx, x_out, idx_v, rows_v, sem):
        w = lax.axis_index("s") * 2 + lax.axis_index("c")
        pltpu.sync_copy(idx.at[w], idx_v)
        for q in range(_NE):
            pltpu.async_copy(tab.at[idx_v.at[q]], rows_v, sem).wait()
            pltpu.sync_copy(rows_v, x_out.at[pl.ds(w * _BPW + q * _KE, _KE)])

    return _sc_embed


@functools.cache
def _sc_edge_kernel():
    mesh = plsc.VectorSubcoreMesh(core_axis_name="c", subcore_axis_name="s")
    scratch_types = [
        pltpu.VMEM((_NCH * _KC,), jnp.int32),
        pltpu.VMEM((_NCH, _KC), jnp.int32),
        [pltpu.VMEM((_KC,), jnp.float32)] * 2,
        [pltpu.VMEM((_KC,), jnp.float32)] * 2,
        pltpu.VMEM((16,), jnp.float32),
        [pltpu.VMEM((_KC, _HC), jnp.float32)] * 3,
        [pltpu.VMEM((_KC,), jnp.float32)] * 2,
        pltpu.VMEM((_RPT,), jnp.float32),
        pltpu.VMEM_SHARED((_NP, _HC), jnp.float32),
        pltpu.VMEM_SHARED((_NP,), jnp.float32),
        [pltpu.SemaphoreType.DMA] * 3,
        [pltpu.SemaphoreType.DMA] * 2,
        [pltpu.SemaphoreType.DMA] * 2,
        [pltpu.SemaphoreType.DMA] * 3,
        [pltpu.SemaphoreType.DMA] * 2,
    ]

    @functools.partial(
        pl.kernel, mesh=mesh,
        out_type=[jax.ShapeDtypeStruct((2, _NP, _HC), jnp.float32),
                  jax.ShapeDtypeStruct((2, _NP), jnp.float32)],
        scratch_types=scratch_types,
        compiler_params=pltpu.CompilerParams(needs_layout_passes=False,
                                             use_tc_tiling_on_sc=False),
    )
    def _sc_edge(srcl, srch, dst, ss2, sd, mg, hil, acc_out, den_out,
                 src_v, dst_v, ssg, sdg, mg_v, rows, ex, zden_v,
                 acc_sh, den_sh, gsem, sssem, sdsem, scsem, dnsem):
        _sc_edge_body(srcl, srch, dst, ss2, sd, mg, hil, acc_out, den_out,
                      src_v, dst_v, ssg, sdg, mg_v, rows, ex, zden_v,
                      acc_sh, den_sh, gsem, sssem, sdsem, scsem, dnsem)

    return _sc_edge


def _sc_edge_body(srcl, srch, dst, ss2, sd, mg, hil, acc_out, den_out,
                  src_v, dst_v, ssg, sdg, mg_v, rows, ex, zden_v,
                  acc_sh, den_sh, gsem, sssem, sdsem, scsem, dnsem):
    c = lax.axis_index("c")
    s = lax.axis_index("s")
    pltpu.sync_copy(mg, mg_v)

    @pl.when(c == 0)
    def _():
        pltpu.sync_copy(srcl.at[s], src_v)

    @pl.when(c == 1)
    def _():
        pltpu.sync_copy(srch.at[s], src_v)
    pltpu.sync_copy(dst.at[s], dst_v)

    zv = jnp.zeros((16,), jnp.float32)

    def _zrow(j, carry):
        for cg in range(_HC // 16):
            rows[0][j, pl.ds(cg * 16, 16)] = zv
        return carry
    lax.fori_loop(0, _KC, _zrow, 0)

    def _zden(i, carry):
        zden_v[pl.ds(pl.multiple_of(i * 16, 16), 16)] = zv
        return carry
    lax.fori_loop(0, _RPT // 16, _zden, 0)

    base = s * _RPT
    for q in range(_RPT // _KC):
        pltpu.sync_copy(rows[0], acc_sh.at[pl.ds(base + q * _KC, _KC)])
    pltpu.sync_copy(zden_v, den_sh.at[pl.ds(base, _RPT)])
    plsc.subcore_barrier()

    mgv = mg_v[...]

    def _sidx(k):
        off = k * _KC if isinstance(k, int) else pl.multiple_of(k * _KC, 8)
        return src_v.at[pl.ds(off, _KC)]

    def _when(cond, fn):
        if isinstance(cond, bool):
            if cond:
                fn()
        else:
            pl.when(cond)(fn)

    def _issue(k, b3, b2):
        pltpu.async_copy(hil.at[_sidx(k)], rows[b3], gsem[b3])
        pltpu.async_copy(ss2.at[_sidx(k)], ssg[b2], sssem[b2])
        pltpu.async_copy(sd.at[dst_v.at[k]], sdg[b2], sdsem[b2])

    def _wait_scatter(k, b3):
        pltpu.make_async_copy(rows[b3], acc_sh.at[dst_v.at[k]],
                              scsem[b3]).wait()

    def _wait_den(k, b2):
        pltpu.make_async_copy(ex[b2], den_sh.at[dst_v.at[k]],
                              dnsem[b2]).wait()

    def _process(k, b3, b2):
        pltpu.make_async_copy(ss2.at[_sidx(k)], ssg[b2], sssem[b2]).wait()
        pltpu.make_async_copy(sd.at[dst_v.at[k]], sdg[b2], sdsem[b2]).wait()

        def _grp(g, carry2):
            off = pl.multiple_of(g * 16, 16)
            av = ssg[b2][pl.ds(off, 16)]
            bv = sdg[b2][pl.ds(off, 16)]
            lg = av + bv
            lg = jnp.maximum(lg, 0.2 * lg)
            mm = mgv + bv
            mm = jnp.maximum(mm, 0.2 * mm)
            ex[b2][pl.ds(off, 16)] = jnp.exp(lg - mm)
            return carry2
        lax.fori_loop(0, _KC // 16, _grp, 0)

        pltpu.make_async_copy(hil.at[_sidx(k)], rows[b3], gsem[b3]).wait()

        def _scale(blk, carry2):
            exs = ex[b2].at[pl.ds(pl.multiple_of(blk * 16, 16), 16)]
            for r in range(16):
                ej = plsc.load_gather(exs, [jnp.full((16,), r, jnp.int32)])
                j = blk * 16 + r
                for cg in range(_HC // 16):
                    sl = pl.ds(cg * 16, 16)
                    rows[b3][j, sl] = rows[b3][j, sl] * ej
            return carry2
        lax.fori_loop(0, _KC // 16, _scale, 0)

        pltpu.async_copy(rows[b3], acc_sh.at[dst_v.at[k]], scsem[b3],
                         add=True)
        pltpu.async_copy(ex[b2], den_sh.at[dst_v.at[k]], dnsem[b2], add=True)

    def _step(k, b3, b2, issue_next):
        if issue_next:
            _when(k >= 2, lambda: _wait_scatter(k - 2, (b3 + 1) % 3))
            _issue(k + 1, (b3 + 1) % 3, (b2 + 1) % 2)

        _when(k >= 2, lambda: _wait_den(k - 2, b2))

        _process(k, b3, b2)

    _issue(0, 0, 0)

    def _macro(jj, carry):
        for i in range(6):
            k = jj * 6 + i
            _step(k, i % 3, i % 2, True)
        return carry
    lax.fori_loop(0, _NCH // 6 - 1, _macro, 0)

    for i in range(6):
        k = _NCH - 6 + i
        _step(k, i % 3, i % 2, issue_next=(i < 5))
    for k in (_NCH - 3, _NCH - 2, _NCH - 1):
        _wait_scatter(k, k % 3)
    for k in (_NCH - 2, _NCH - 1):
        _wait_den(k, k % 2)
    plsc.subcore_barrier()

    pltpu.sync_copy(acc_sh.at[pl.ds(base, _RPT)],
                    acc_out.at[c, pl.ds(base, _RPT)])
    pltpu.sync_copy(den_sh.at[pl.ds(base, _RPT)],
                    den_out.at[c, pl.ds(base, _RPT)])


def _t1_body(x_ref, w_ref, as_ref, ad_ref, h_ref, ss_ref, sd_ref, mg_ref):
    h = jnp.dot(x_ref[...], w_ref[...], preferred_element_type=jnp.float32)
    h_ref[...] = h
    ss = jnp.dot(h, as_ref[...], preferred_element_type=jnp.float32)
    sd = jnp.dot(h, ad_ref[...], preferred_element_type=jnp.float32)
    ss_ref[...] = ss
    sd_ref[...] = sd

    @pl.when(pl.program_id(0) == 0)
    def _():
        mg_ref[...] = jnp.full((1, 1), -1e30, jnp.float32)
    mg_ref[...] = jnp.maximum(mg_ref[...], jnp.max(ss))


def _t2_body(acc_ref, den_ref, b_ref, w_ref, as_ref, ad_ref,
             h_ref, ss_ref, sd_ref, mg_ref):
    den = den_ref[0]
    den = jnp.where(den == 0.0, 1.0, den)
    acc = jnp.concatenate([acc_ref[0], acc_ref[1]], axis=1)
    x = jnp.maximum(acc / den + b_ref[...], 0.0)
    h = jnp.dot(x, w_ref[...], preferred_element_type=jnp.float32)
    h_ref[...] = h
    ss = jnp.dot(h, as_ref[...], preferred_element_type=jnp.float32)
    sd = jnp.dot(h, ad_ref[...], preferred_element_type=jnp.float32)
    ss_ref[...] = ss
    sd_ref[...] = sd

    @pl.when(pl.program_id(0) == 0)
    def _():
        mg_ref[...] = jnp.full((1, 1), -1e30, jnp.float32)
    mg_ref[...] = jnp.maximum(mg_ref[...], jnp.max(ss))


def _t3_body(acc_ref, den_ref, b_ref, gw_ref, gb_ref, ids_ref, out_ref):
    den = den_ref[0]
    den = jnp.where(den == 0.0, 1.0, den)
    acc = jnp.concatenate([acc_ref[0], acc_ref[1]], axis=1)
    x = jnp.maximum(acc / den + b_ref[...], 0.0)
    g = jnp.dot(x, gw_ref[...], preferred_element_type=jnp.float32) + gb_ref[...]
    cols = lax.broadcasted_iota(jnp.int32, (_N, _H), 1)
    mask = ids_ref[...] == cols
    gm = jnp.max(jnp.where(mask, g, -1e30), axis=0, keepdims=True)
    mn = jnp.sum(jnp.where(mask, gm, 0.0), axis=1, keepdims=True)
    e = jnp.exp(g - mn)
    dsum = jnp.sum(jnp.where(mask, e, 0.0), axis=0, keepdims=True)
    dn = jnp.sum(jnp.where(mask, dsum, 0.0), axis=1, keepdims=True)
    wgt = x * (e / dn)
    res = lax.dot_general(mask.astype(jnp.float32), wgt,
                          (((0,), (0,)), ((), ())),
                          preferred_element_type=jnp.float32)
    out_ref[...] = res[:_G, :]


_R = 2048


def _tc_dense1(x, W, a_s, a_d):
    return pl.pallas_call(
        _t1_body,
        grid=(_NP // _R,),
        in_specs=[
            pl.BlockSpec((_R, _H), lambda i: (i, 0)),
            pl.BlockSpec((_H, _H), lambda i: (0, 0)),
            pl.BlockSpec((_H, 1), lambda i: (0, 0)),
            pl.BlockSpec((_H, 1), lambda i: (0, 0)),
        ],
        out_specs=[
            pl.BlockSpec((_R, _H), lambda i: (i, 0)),
            pl.BlockSpec((_R, 1), lambda i: (i, 0)),
            pl.BlockSpec((_R, 1), lambda i: (i, 0)),
            pl.BlockSpec((1, 1), lambda i: (0, 0)),
        ],
        out_shape=[
            jax.ShapeDtypeStruct((_NP, _H), jnp.float32),
            jax.ShapeDtypeStruct((_NP, 1), jnp.float32),
            jax.ShapeDtypeStruct((_NP, 1), jnp.float32),
            jax.ShapeDtypeStruct((1, 1), jnp.float32),
        ],
        compiler_params=pltpu.CompilerParams(
            dimension_semantics=("arbitrary",)),
    )(x, W, a_s, a_d)


def _tc_dense2(acc, den, b, W, a_s, a_d):
    return pl.pallas_call(
        _t2_body,
        grid=(_NP // _R,),
        in_specs=[
            pl.BlockSpec((2, _R, _HC), lambda i: (0, i, 0)),
            pl.BlockSpec((2, _R, 1), lambda i: (0, i, 0)),
            pl.BlockSpec((1, _H), lambda i: (0, 0)),
            pl.BlockSpec((_H, _H), lambda i: (0, 0)),
            pl.BlockSpec((_H, 1), lambda i: (0, 0)),
            pl.BlockSpec((_H, 1), lambda i: (0, 0)),
        ],
        out_specs=[
            pl.BlockSpec((_R, _H), lambda i: (i, 0)),
            pl.BlockSpec((_R, 1), lambda i: (i, 0)),
            pl.BlockSpec((_R, 1), lambda i: (i, 0)),
            pl.BlockSpec((1, 1), lambda i: (0, 0)),
        ],
        out_shape=[
            jax.ShapeDtypeStruct((_NP, _H), jnp.float32),
            jax.ShapeDtypeStruct((_NP, 1), jnp.float32),
            jax.ShapeDtypeStruct((_NP, 1), jnp.float32),
            jax.ShapeDtypeStruct((1, 1), jnp.float32),
        ],
        compiler_params=pltpu.CompilerParams(
            dimension_semantics=("arbitrary",)),
    )(acc, den, b, W, a_s, a_d)


def _tc_final(acc, den, b, gw, gb, ids):
    return pl.pallas_call(
        _t3_body,
        out_shape=jax.ShapeDtypeStruct((_G, _H), jnp.float32),
    )(acc, den, b, gw, gb, ids)


def kernel(input_ids, attention_mask, edge_index, input_ids_batch, embed_table,
           W1, a_src1, a_dst1, b1, W2, a_src2, a_dst2, b2, gate_W, gate_b):
    f32 = jnp.float32
    last = input_ids[:, -1].astype(jnp.int32)
    idx3 = jnp.concatenate(
        [last, jnp.zeros((_NP - _N,), jnp.int32)]).reshape(_NW, _NE, _KE)
    x = _sc_embed_kernel()(embed_table.astype(f32), idx3)

    loop = jnp.arange(_N, dtype=jnp.int32)
    padn = _EP - _E - _N
    pad_src = jnp.arange(padn, dtype=jnp.int32) % 240
    pad_dst = _N + jnp.arange(padn, dtype=jnp.int32) % 240
    esrc = jnp.concatenate([edge_index[0].astype(jnp.int32), loop, pad_src])
    edst = jnp.concatenate([edge_index[1].astype(jnp.int32), loop, pad_dst])
    srcl = (2 * esrc).reshape(16, _NCH * _KC)
    srch = (2 * esrc + 1).reshape(16, _NCH * _KC)
    dst3 = edst.reshape(16, _NCH, _KC)

    def _layer(h, ssx, sdx, mgx):
        return _sc_edge_kernel()(
            srcl, srch, dst3, jnp.repeat(ssx.reshape(_NP), 2),
            sdx.reshape(_NP), jnp.broadcast_to(mgx[0, 0], (16,)),
            h.reshape(2 * _NP, _HC))

    h1, ss1, sd1, mg1 = _tc_dense1(
        x, W1, a_src1.reshape(_H, 1), a_dst1.reshape(_H, 1))
    acc1, den1 = _layer(h1, ss1, sd1, mg1)
    h2, ss2, sd2, mg2 = _tc_dense2(
        acc1, den1.reshape(2, _NP, 1), b1.reshape(1, _H),
        W2, a_src2.reshape(_H, 1), a_dst2.reshape(_H, 1))
    acc2, den2 = _layer(h2, ss2, sd2, mg2)
    out = _tc_final(acc2[:, :_N, :], den2[:, :_N].reshape(2, _N, 1),
                    b2.reshape(1, _H), gate_W, gate_b.reshape(1, 1),
                    input_ids_batch.astype(jnp.int32).reshape(_N, 1))
    return out


# speedup vs baseline: 1.0355x; 1.0355x over previous
import functools

import jax
import jax.numpy as jnp
from jax import lax
from jax.experimental import pallas as pl
from jax.experimental.pallas import tpu as pltpu
from jax.experimental.pallas import tpu_sc as plsc

_N = 10000
_H = 128
_G = 64
_NP = 10240
_E = 320000
_EP = 331776
_NW = 32
_HC = _H // 2
_KC = 128
_NCH = _EP // 16 // _KC
_RPT = _NP // 16
_BPW = _NP // _NW
_KE = 64
_NE = _BPW // _KE

@functools.cache
def _sc_embed_kernel():
    mesh = plsc.VectorSubcoreMesh(core_axis_name="c", subcore_axis_name="s")

    @functools.partial(
        pl.kernel, mesh=mesh,
        out_type=jax.ShapeDtypeStruct((_NP, _H), jnp.float32),
        scratch_types=[
            pltpu.VMEM((_NE, _KE), jnp.int32),
            [pltpu.VMEM((_KE, _H), jnp.float32)] * 2,
            [pltpu.SemaphoreType.DMA] * 2,
            [pltpu.SemaphoreType.DMA] * 2,
        ],
        compiler_params=pltpu.CompilerParams(needs_layout_passes=False),
    )
    def _sc_embed(tab, idx, x_out, idx_v, rows, gsem, ssem):
        w = lax.axis_index("s") * 2 + lax.axis_index("c")
        pltpu.sync_copy(idx.at[w], idx_v)

        def _out_at(q):
            return x_out.at[pl.ds(w * _BPW + q * _KE, _KE)]

        pltpu.async_copy(tab.at[idx_v.at[0]], rows[0], gsem[0])
        for q in range(_NE):
            b, nb = q % 2, (q + 1) % 2
            if q + 1 < _NE:
                if q >= 1:
                    pltpu.make_async_copy(rows[nb], _out_at(q - 1),
                                          ssem[nb]).wait()
                pltpu.async_copy(tab.at[idx_v.at[q + 1]], rows[nb], gsem[nb])
            pltpu.make_async_copy(tab.at[idx_v.at[q]], rows[b],
                                  gsem[b]).wait()
            pltpu.async_copy(rows[b], _out_at(q), ssem[b])
        for q in range(_NE - 2, _NE):
            pltpu.make_async_copy(rows[q % 2], _out_at(q),
                                  ssem[q % 2]).wait()

    return _sc_embed


@functools.cache
def _sc_edge_kernel():
    mesh = plsc.VectorSubcoreMesh(core_axis_name="c", subcore_axis_name="s")
    scratch_types = [
        pltpu.VMEM((_NCH * _KC,), jnp.int32),
        pltpu.VMEM((_NCH, _KC), jnp.int32),
        [pltpu.VMEM((_KC,), jnp.float32)] * 2,
        [pltpu.VMEM((_KC,), jnp.float32)] * 2,
        pltpu.VMEM((16,), jnp.float32),
        [pltpu.VMEM((_KC, _HC), jnp.float32)] * 3,
        [pltpu.VMEM((_KC,), jnp.float32)] * 2,
        pltpu.VMEM((_RPT,), jnp.float32),
        pltpu.VMEM_SHARED((_NP, _HC), jnp.float32),
        pltpu.VMEM_SHARED((_NP,), jnp.float32),
        [pltpu.SemaphoreType.DMA] * 3,
        [pltpu.SemaphoreType.DMA] * 2,
        [pltpu.SemaphoreType.DMA] * 2,
        [pltpu.SemaphoreType.DMA] * 3,
        [pltpu.SemaphoreType.DMA] * 2,
    ]

    @functools.partial(
        pl.kernel, mesh=mesh,
        out_type=[jax.ShapeDtypeStruct((2, _NP, _HC), jnp.float32),
                  jax.ShapeDtypeStruct((2, _NP), jnp.float32)],
        scratch_types=scratch_types,
        compiler_params=pltpu.CompilerParams(needs_layout_passes=False,
                                             use_tc_tiling_on_sc=False),
    )
    def _sc_edge(srcl, srch, dst, ss2, sd, mg, hil, acc_out, den_out,
                 src_v, dst_v, ssg, sdg, mg_v, rows, ex, zden_v,
                 acc_sh, den_sh, gsem, sssem, sdsem, scsem, dnsem):
        _sc_edge_body(srcl, srch, dst, ss2, sd, mg, hil, acc_out, den_out,
                      src_v, dst_v, ssg, sdg, mg_v, rows, ex, zden_v,
                      acc_sh, den_sh, gsem, sssem, sdsem, scsem, dnsem)

    return _sc_edge


def _sc_edge_body(srcl, srch, dst, ss2, sd, mg, hil, acc_out, den_out,
                  src_v, dst_v, ssg, sdg, mg_v, rows, ex, zden_v,
                  acc_sh, den_sh, gsem, sssem, sdsem, scsem, dnsem):
    c = lax.axis_index("c")
    s = lax.axis_index("s")
    pltpu.sync_copy(mg, mg_v)

    @pl.when(c == 0)
    def _():
        pltpu.sync_copy(srcl.at[s], src_v)

    @pl.when(c == 1)
    def _():
        pltpu.sync_copy(srch.at[s], src_v)
    pltpu.sync_copy(dst.at[s], dst_v)

    zv = jnp.zeros((16,), jnp.float32)

    def _zrow(j, carry):
        for cg in range(_HC // 16):
            rows[0][j, pl.ds(cg * 16, 16)] = zv
        return carry
    lax.fori_loop(0, _KC, _zrow, 0)

    def _zden(i, carry):
        zden_v[pl.ds(pl.multiple_of(i * 16, 16), 16)] = zv
        return carry
    lax.fori_loop(0, _RPT // 16, _zden, 0)

    base = s * _RPT
    for q in range(_RPT // _KC):
        pltpu.sync_copy(rows[0], acc_sh.at[pl.ds(base + q * _KC, _KC)])
    pltpu.sync_copy(zden_v, den_sh.at[pl.ds(base, _RPT)])
    plsc.subcore_barrier()

    mgv = mg_v[...]

    def _sidx(k):
        off = k * _KC if isinstance(k, int) else pl.multiple_of(k * _KC, 8)
        return src_v.at[pl.ds(off, _KC)]

    def _when(cond, fn):
        if isinstance(cond, bool):
            if cond:
                fn()
        else:
            pl.when(cond)(fn)

    def _issue(k, b3, b2):
        pltpu.async_copy(hil.at[_sidx(k)], rows[b3], gsem[b3])
        pltpu.async_copy(ss2.at[_sidx(k)], ssg[b2], sssem[b2])
        pltpu.async_copy(sd.at[dst_v.at[k]], sdg[b2], sdsem[b2])

    def _wait_scatter(k, b3):
        pltpu.make_async_copy(rows[b3], acc_sh.at[dst_v.at[k]],
                              scsem[b3]).wait()

    def _wait_den(k, b2):
        pltpu.make_async_copy(ex[b2], den_sh.at[dst_v.at[k]],
                              dnsem[b2]).wait()

    def _process(k, b3, b2):
        pltpu.make_async_copy(ss2.at[_sidx(k)], ssg[b2], sssem[b2]).wait()
        pltpu.make_async_copy(sd.at[dst_v.at[k]], sdg[b2], sdsem[b2]).wait()

        def _grp(g, carry2):
            off = pl.multiple_of(g * 16, 16)
            av = ssg[b2][pl.ds(off, 16)]
            bv = sdg[b2][pl.ds(off, 16)]
            lg = av + bv
            lg = jnp.maximum(lg, 0.2 * lg)
            mm = mgv + bv
            mm = jnp.maximum(mm, 0.2 * mm)
            ex[b2][pl.ds(off, 16)] = jnp.exp(lg - mm)
            return carry2
        lax.fori_loop(0, _KC // 16, _grp, 0)

        pltpu.make_async_copy(hil.at[_sidx(k)], rows[b3], gsem[b3]).wait()

        def _scale(jj, carry2):
            for u in range(2):
                j = jj * 2 + u
                ej = plsc.load_gather(ex[b2],
                                      [jnp.zeros((16,), jnp.int32) + j])
                for cg in range(_HC // 16):
                    sl = pl.ds(cg * 16, 16)
                    rows[b3][j, sl] = rows[b3][j, sl] * ej
            return carry2
        lax.fori_loop(0, _KC // 2, _scale, 0)

        pltpu.async_copy(rows[b3], acc_sh.at[dst_v.at[k]], scsem[b3],
                         add=True)
        pltpu.async_copy(ex[b2], den_sh.at[dst_v.at[k]], dnsem[b2], add=True)

    def _step(k, b3, b2, issue_next):
        if issue_next:
            _when(k >= 2, lambda: _wait_scatter(k - 2, (b3 + 1) % 3))
            _issue(k + 1, (b3 + 1) % 3, (b2 + 1) % 2)

        _when(k >= 2, lambda: _wait_den(k - 2, b2))

        _process(k, b3, b2)

    _issue(0, 0, 0)

    def _macro(jj, carry):
        for i in range(6):
            k = jj * 6 + i
            _step(k, i % 3, i % 2, True)
        return carry
    lax.fori_loop(0, _NCH // 6 - 1, _macro, 0)

    for i in range(6):
        k = _NCH - 6 + i
        _step(k, i % 3, i % 2, issue_next=(i < 5))
    for k in (_NCH - 3, _NCH - 2, _NCH - 1):
        _wait_scatter(k, k % 3)
    for k in (_NCH - 2, _NCH - 1):
        _wait_den(k, k % 2)
    plsc.subcore_barrier()

    pltpu.sync_copy(acc_sh.at[pl.ds(base, _RPT)],
                    acc_out.at[c, pl.ds(base, _RPT)])
    pltpu.sync_copy(den_sh.at[pl.ds(base, _RPT)],
                    den_out.at[c, pl.ds(base, _RPT)])


def _t1_body(x_ref, w_ref, as_ref, ad_ref, h_ref, ss_ref, sd_ref, mg_ref):
    h = jnp.dot(x_ref[...], w_ref[...], preferred_element_type=jnp.float32)
    h_ref[...] = h
    ss = jnp.dot(h, as_ref[...], preferred_element_type=jnp.float32)
    sd = jnp.dot(h, ad_ref[...], preferred_element_type=jnp.float32)
    ss_ref[...] = ss
    sd_ref[...] = sd

    @pl.when(pl.program_id(0) == 0)
    def _():
        mg_ref[...] = jnp.full((1, 1), -1e30, jnp.float32)
    mg_ref[...] = jnp.maximum(mg_ref[...], jnp.max(ss))


def _t2_body(acc_ref, den_ref, b_ref, w_ref, as_ref, ad_ref,
             h_ref, ss_ref, sd_ref, mg_ref):
    den = den_ref[0]
    den = jnp.where(den == 0.0, 1.0, den)
    acc = jnp.concatenate([acc_ref[0], acc_ref[1]], axis=1)
    x = jnp.maximum(acc / den + b_ref[...], 0.0)
    h = jnp.dot(x, w_ref[...], preferred_element_type=jnp.float32)
    h_ref[...] = h
    ss = jnp.dot(h, as_ref[...], preferred_element_type=jnp.float32)
    sd = jnp.dot(h, ad_ref[...], preferred_element_type=jnp.float32)
    ss_ref[...] = ss
    sd_ref[...] = sd

    @pl.when(pl.program_id(0) == 0)
    def _():
        mg_ref[...] = jnp.full((1, 1), -1e30, jnp.float32)
    mg_ref[...] = jnp.maximum(mg_ref[...], jnp.max(ss))


def _t3_body(acc_ref, den_ref, b_ref, gw_ref, gb_ref, ids_ref, out_ref):
    den = den_ref[0]
    den = jnp.where(den == 0.0, 1.0, den)
    acc = jnp.concatenate([acc_ref[0], acc_ref[1]], axis=1)
    x = jnp.maximum(acc / den + b_ref[...], 0.0)
    g = jnp.dot(x, gw_ref[...], preferred_element_type=jnp.float32) + gb_ref[...]
    cols = lax.broadcasted_iota(jnp.int32, (_N, _H), 1)
    mask = ids_ref[...] == cols
    gm = jnp.max(jnp.where(mask, g, -1e30), axis=0, keepdims=True)
    mn = jnp.sum(jnp.where(mask, gm, 0.0), axis=1, keepdims=True)
    e = jnp.exp(g - mn)
    dsum = jnp.sum(jnp.where(mask, e, 0.0), axis=0, keepdims=True)
    dn = jnp.sum(jnp.where(mask, dsum, 0.0), axis=1, keepdims=True)
    wgt = x * (e / dn)
    res = lax.dot_general(mask.astype(jnp.float32), wgt,
                          (((0,), (0,)), ((), ())),
                          preferred_element_type=jnp.float32)
    out_ref[...] = res[:_G, :]


_R = 2048


def _tc_dense1(x, W, a_s, a_d):
    return pl.pallas_call(
        _t1_body,
        grid=(_NP // _R,),
        in_specs=[
            pl.BlockSpec((_R, _H), lambda i: (i, 0)),
            pl.BlockSpec((_H, _H), lambda i: (0, 0)),
            pl.BlockSpec((_H, 1), lambda i: (0, 0)),
            pl.BlockSpec((_H, 1), lambda i: (0, 0)),
        ],
        out_specs=[
            pl.BlockSpec((_R, _H), lambda i: (i, 0)),
            pl.BlockSpec((_R, 1), lambda i: (i, 0)),
            pl.BlockSpec((_R, 1), lambda i: (i, 0)),
            pl.BlockSpec((1, 1), lambda i: (0, 0)),
        ],
        out_shape=[
            jax.ShapeDtypeStruct((_NP, _H), jnp.float32),
            jax.ShapeDtypeStruct((_NP, 1), jnp.float32),
            jax.ShapeDtypeStruct((_NP, 1), jnp.float32),
            jax.ShapeDtypeStruct((1, 1), jnp.float32),
        ],
        compiler_params=pltpu.CompilerParams(
            dimension_semantics=("arbitrary",)),
    )(x, W, a_s, a_d)


def _tc_dense2(acc, den, b, W, a_s, a_d):
    return pl.pallas_call(
        _t2_body,
        grid=(_NP // _R,),
        in_specs=[
            pl.BlockSpec((2, _R, _HC), lambda i: (0, i, 0)),
            pl.BlockSpec((2, _R, 1), lambda i: (0, i, 0)),
            pl.BlockSpec((1, _H), lambda i: (0, 0)),
            pl.BlockSpec((_H, _H), lambda i: (0, 0)),
            pl.BlockSpec((_H, 1), lambda i: (0, 0)),
            pl.BlockSpec((_H, 1), lambda i: (0, 0)),
        ],
        out_specs=[
            pl.BlockSpec((_R, _H), lambda i: (i, 0)),
            pl.BlockSpec((_R, 1), lambda i: (i, 0)),
            pl.BlockSpec((_R, 1), lambda i: (i, 0)),
            pl.BlockSpec((1, 1), lambda i: (0, 0)),
        ],
        out_shape=[
            jax.ShapeDtypeStruct((_NP, _H), jnp.float32),
            jax.ShapeDtypeStruct((_NP, 1), jnp.float32),
            jax.ShapeDtypeStruct((_NP, 1), jnp.float32),
            jax.ShapeDtypeStruct((1, 1), jnp.float32),
        ],
        compiler_params=pltpu.CompilerParams(
            dimension_semantics=("arbitrary",)),
    )(acc, den, b, W, a_s, a_d)


def _tc_final(acc, den, b, gw, gb, ids):
    return pl.pallas_call(
        _t3_body,
        out_shape=jax.ShapeDtypeStruct((_G, _H), jnp.float32),
    )(acc, den, b, gw, gb, ids)


def kernel(input_ids, attention_mask, edge_index, input_ids_batch, embed_table,
           W1, a_src1, a_dst1, b1, W2, a_src2, a_dst2, b2, gate_W, gate_b):
    f32 = jnp.float32
    last = input_ids[:, -1].astype(jnp.int32)
    idx3 = jnp.concatenate(
        [last, jnp.zeros((_NP - _N,), jnp.int32)]).reshape(_NW, _NE, _KE)
    x = _sc_embed_kernel()(embed_table.astype(f32), idx3)

    loop = jnp.arange(_N, dtype=jnp.int32)
    padn = _EP - _E - _N
    pad_src = jnp.arange(padn, dtype=jnp.int32) % 240
    pad_dst = _N + jnp.arange(padn, dtype=jnp.int32) % 240
    esrc = jnp.concatenate([edge_index[0].astype(jnp.int32), loop, pad_src])
    edst = jnp.concatenate([edge_index[1].astype(jnp.int32), loop, pad_dst])
    srcl = (2 * esrc).reshape(16, _NCH * _KC)
    srch = (2 * esrc + 1).reshape(16, _NCH * _KC)
    dst3 = edst.reshape(16, _NCH, _KC)

    def _layer(h, ssx, sdx, mgx):
        return _sc_edge_kernel()(
            srcl, srch, dst3, jnp.repeat(ssx.reshape(_NP), 2),
            sdx.reshape(_NP), jnp.broadcast_to(mgx[0, 0], (16,)),
            h.reshape(2 * _NP, _HC))

    h1, ss1, sd1, mg1 = _tc_dense1(
        x, W1, a_src1.reshape(_H, 1), a_dst1.reshape(_H, 1))
    acc1, den1 = _layer(h1, ss1, sd1, mg1)
    h2, ss2, sd2, mg2 = _tc_dense2(
        acc1, den1.reshape(2, _NP, 1), b1.reshape(1, _H),
        W2, a_src2.reshape(_H, 1), a_dst2.reshape(_H, 1))
    acc2, den2 = _layer(h2, ss2, sd2, mg2)
    out = _tc_final(acc2[:, :_N, :], den2[:, :_N].reshape(2, _N, 1),
                    b2.reshape(1, _H), gate_W, gate_b.reshape(1, 1),
                    input_ids_batch.astype(jnp.int32).reshape(_N, 1))
    return out


# speedup vs baseline: 1.0363x; 1.0008x over previous
import functools

import jax
import jax.numpy as jnp
from jax import lax
from jax.experimental import pallas as pl
from jax.experimental.pallas import tpu as pltpu
from jax.experimental.pallas import tpu_sc as plsc

_N = 10000
_H = 128
_G = 64
_NP = 10240
_E = 320000
_EP = 331776
_NW = 32
_HC = _H // 2
_KC = 128
_NCH = _EP // 16 // _KC
_RPT = _NP // 16
_BPW = _NP // _NW
_KE = 64
_NE = _BPW // _KE

@functools.cache
def _sc_embed_kernel():
    mesh = plsc.VectorSubcoreMesh(core_axis_name="c", subcore_axis_name="s")

    @functools.partial(
        pl.kernel, mesh=mesh,
        out_type=jax.ShapeDtypeStruct((_NP, _H), jnp.float32),
        scratch_types=[
            pltpu.VMEM((_NE, _KE), jnp.int32),
            [pltpu.VMEM((_KE, _H), jnp.float32)] * 2,
            [pltpu.SemaphoreType.DMA] * 2,
            [pltpu.SemaphoreType.DMA] * 2,
        ],
        compiler_params=pltpu.CompilerParams(needs_layout_passes=False),
    )
    def _sc_embed(tab, idx, x_out, idx_v, rows, gsem, ssem):
        w = lax.axis_index("s") * 2 + lax.axis_index("c")
        pltpu.sync_copy(idx.at[w], idx_v)

        def _out_at(q):
            return x_out.at[pl.ds(w * _BPW + q * _KE, _KE)]

        pltpu.async_copy(tab.at[idx_v.at[0]], rows[0], gsem[0])
        for q in range(_NE):
            b, nb = q % 2, (q + 1) % 2
            if q + 1 < _NE:
                if q >= 1:
                    pltpu.make_async_copy(rows[nb], _out_at(q - 1),
                                          ssem[nb]).wait()
                pltpu.async_copy(tab.at[idx_v.at[q + 1]], rows[nb], gsem[nb])
            pltpu.make_async_copy(tab.at[idx_v.at[q]], rows[b],
                                  gsem[b]).wait()
            pltpu.async_copy(rows[b], _out_at(q), ssem[b])
        for q in range(_NE - 2, _NE):
            pltpu.make_async_copy(rows[q % 2], _out_at(q),
                                  ssem[q % 2]).wait()

    return _sc_embed


@functools.cache
def _sc_edge_kernel():
    mesh = plsc.VectorSubcoreMesh(core_axis_name="c", subcore_axis_name="s")
    scratch_types = [
        pltpu.VMEM((_NCH * _KC,), jnp.int32),
        pltpu.VMEM((_NCH, _KC), jnp.int32),
        [pltpu.VMEM((_KC,), jnp.float32)] * 2,
        [pltpu.VMEM((_KC,), jnp.float32)] * 2,
        pltpu.VMEM((16,), jnp.float32),
        [pltpu.VMEM((_KC, _HC), jnp.float32)] * 3,
        [pltpu.VMEM((_KC,), jnp.float32)] * 2,
        pltpu.VMEM((_RPT,), jnp.float32),
        pltpu.VMEM_SHARED((_NP, _HC), jnp.float32),
        pltpu.VMEM_SHARED((_NP,), jnp.float32),
        [pltpu.SemaphoreType.DMA] * 3,
        [pltpu.SemaphoreType.DMA] * 2,
        [pltpu.SemaphoreType.DMA] * 2,
        [pltpu.SemaphoreType.DMA] * 3,
        [pltpu.SemaphoreType.DMA] * 2,
    ]

    @functools.partial(
        pl.kernel, mesh=mesh,
        out_type=[jax.ShapeDtypeStruct((2, _NP, _HC), jnp.float32),
                  jax.ShapeDtypeStruct((2, _NP), jnp.float32)],
        scratch_types=scratch_types,
        compiler_params=pltpu.CompilerParams(needs_layout_passes=False,
                                             use_tc_tiling_on_sc=False),
    )
    def _sc_edge(srcl, srch, dst, ss2, sd, mg, hil, acc_out, den_out,
                 src_v, dst_v, ssg, sdg, mg_v, rows, ex, zden_v,
                 acc_sh, den_sh, gsem, sssem, sdsem, scsem, dnsem):
        _sc_edge_body(srcl, srch, dst, ss2, sd, mg, hil, acc_out, den_out,
                      src_v, dst_v, ssg, sdg, mg_v, rows, ex, zden_v,
                      acc_sh, den_sh, gsem, sssem, sdsem, scsem, dnsem)

    return _sc_edge


def _sc_edge_body(srcl, srch, dst, ss2, sd, mg, hil, acc_out, den_out,
                  src_v, dst_v, ssg, sdg, mg_v, rows, ex, zden_v,
                  acc_sh, den_sh, gsem, sssem, sdsem, scsem, dnsem):
    c = lax.axis_index("c")
    s = lax.axis_index("s")
    pltpu.sync_copy(mg, mg_v)

    @pl.when(c == 0)
    def _():
        pltpu.sync_copy(srcl.at[s], src_v)

    @pl.when(c == 1)
    def _():
        pltpu.sync_copy(srch.at[s], src_v)
    pltpu.sync_copy(dst.at[s], dst_v)

    zv = jnp.zeros((16,), jnp.float32)

    def _zrow(j, carry):
        for cg in range(_HC // 16):
            rows[0][j, pl.ds(cg * 16, 16)] = zv
        return carry
    lax.fori_loop(0, _KC, _zrow, 0)

    def _zden(i, carry):
        zden_v[pl.ds(pl.multiple_of(i * 16, 16), 16)] = zv
        return carry
    lax.fori_loop(0, _RPT // 16, _zden, 0)

    base = s * _RPT
    for q in range(_RPT // _KC):
        pltpu.sync_copy(rows[0], acc_sh.at[pl.ds(base + q * _KC, _KC)])
    pltpu.sync_copy(zden_v, den_sh.at[pl.ds(base, _RPT)])
    plsc.subcore_barrier()

    mgv = mg_v[...]

    def _sidx(k):
        off = k * _KC if isinstance(k, int) else pl.multiple_of(k * _KC, 8)
        return src_v.at[pl.ds(off, _KC)]

    def _when(cond, fn):
        if isinstance(cond, bool):
            if cond:
                fn()
        else:
            pl.when(cond)(fn)

    def _issue(k, b3, b2):
        pltpu.async_copy(hil.at[_sidx(k)], rows[b3], gsem[b3])
        pltpu.async_copy(ss2.at[_sidx(k)], ssg[b2], sssem[b2])
        pltpu.async_copy(sd.at[dst_v.at[k]], sdg[b2], sdsem[b2])

    def _wait_scatter(k, b3):
        pltpu.make_async_copy(rows[b3], acc_sh.at[dst_v.at[k]],
                              scsem[b3]).wait()

    def _wait_den(k, b2):
        pltpu.make_async_copy(ex[b2], den_sh.at[dst_v.at[k]],
                              dnsem[b2]).wait()

    def _process(k, b3, b2):
        pltpu.make_async_copy(ss2.at[_sidx(k)], ssg[b2], sssem[b2]).wait()
        pltpu.make_async_copy(sd.at[dst_v.at[k]], sdg[b2], sdsem[b2]).wait()

        def _grp(g, carry2):
            off = pl.multiple_of(g * 16, 16)
            av = ssg[b2][pl.ds(off, 16)]
            bv = sdg[b2][pl.ds(off, 16)]
            lg = av + bv
            lg = jnp.maximum(lg, 0.2 * lg)
            mm = mgv + bv
            mm = jnp.maximum(mm, 0.2 * mm)
            ex[b2][pl.ds(off, 16)] = jnp.exp(lg - mm)
            return carry2
        lax.fori_loop(0, _KC // 16, _grp, 0)
        pltpu.async_copy(ex[b2], den_sh.at[dst_v.at[k]], dnsem[b2], add=True)

        pltpu.make_async_copy(hil.at[_sidx(k)], rows[b3], gsem[b3]).wait()

        def _scale(jj, carry2):
            for u in range(2):
                j = jj * 2 + u
                ej = plsc.load_gather(ex[b2],
                                      [jnp.zeros((16,), jnp.int32) + j])
                for cg in range(_HC // 16):
                    sl = pl.ds(cg * 16, 16)
                    rows[b3][j, sl] = rows[b3][j, sl] * ej
            return carry2
        lax.fori_loop(0, _KC // 2, _scale, 0)

        pltpu.async_copy(rows[b3], acc_sh.at[dst_v.at[k]], scsem[b3],
                         add=True)

    def _step(k, b3, b2, issue_next):
        if issue_next:
            _when(k >= 2, lambda: _wait_scatter(k - 2, (b3 + 1) % 3))
            _issue(k + 1, (b3 + 1) % 3, (b2 + 1) % 2)

        _when(k >= 2, lambda: _wait_den(k - 2, b2))

        _process(k, b3, b2)

    _issue(0, 0, 0)

    def _macro(jj, carry):
        for i in range(6):
            k = jj * 6 + i
            _step(k, i % 3, i % 2, True)
        return carry
    lax.fori_loop(0, _NCH // 6 - 1, _macro, 0)

    for i in range(6):
        k = _NCH - 6 + i
        _step(k, i % 3, i % 2, issue_next=(i < 5))
    for k in (_NCH - 3, _NCH - 2, _NCH - 1):
        _wait_scatter(k, k % 3)
    for k in (_NCH - 2, _NCH - 1):
        _wait_den(k, k % 2)
    plsc.subcore_barrier()

    pltpu.sync_copy(acc_sh.at[pl.ds(base, _RPT)],
                    acc_out.at[c, pl.ds(base, _RPT)])
    pltpu.sync_copy(den_sh.at[pl.ds(base, _RPT)],
                    den_out.at[c, pl.ds(base, _RPT)])


def _t1_body(x_ref, w_ref, as_ref, ad_ref, h_ref, ss_ref, sd_ref, mg_ref):
    h = jnp.dot(x_ref[...], w_ref[...], preferred_element_type=jnp.float32)
    h_ref[...] = h
    ss = jnp.dot(h, as_ref[...], preferred_element_type=jnp.float32)
    sd = jnp.dot(h, ad_ref[...], preferred_element_type=jnp.float32)
    ss_ref[...] = ss
    sd_ref[...] = sd

    @pl.when(pl.program_id(0) == 0)
    def _():
        mg_ref[...] = jnp.full((1, 1), -1e30, jnp.float32)
    mg_ref[...] = jnp.maximum(mg_ref[...], jnp.max(ss))


def _t2_body(acc_ref, den_ref, b_ref, w_ref, as_ref, ad_ref,
             h_ref, ss_ref, sd_ref, mg_ref):
    den = den_ref[0]
    den = jnp.where(den == 0.0, 1.0, den)
    acc = jnp.concatenate([acc_ref[0], acc_ref[1]], axis=1)
    x = jnp.maximum(acc / den + b_ref[...], 0.0)
    h = jnp.dot(x, w_ref[...], preferred_element_type=jnp.float32)
    h_ref[...] = h
    ss = jnp.dot(h, as_ref[...], preferred_element_type=jnp.float32)
    sd = jnp.dot(h, ad_ref[...], preferred_element_type=jnp.float32)
    ss_ref[...] = ss
    sd_ref[...] = sd

    @pl.when(pl.program_id(0) == 0)
    def _():
        mg_ref[...] = jnp.full((1, 1), -1e30, jnp.float32)
    mg_ref[...] = jnp.maximum(mg_ref[...], jnp.max(ss))


def _t3_body(acc_ref, den_ref, b_ref, gw_ref, gb_ref, ids_ref, out_ref):
    den = den_ref[0]
    den = jnp.where(den == 0.0, 1.0, den)
    acc = jnp.concatenate([acc_ref[0], acc_ref[1]], axis=1)
    x = jnp.maximum(acc / den + b_ref[...], 0.0)
    g = jnp.dot(x, gw_ref[...], preferred_element_type=jnp.float32) + gb_ref[...]
    cols = lax.broadcasted_iota(jnp.int32, (_N, _H), 1)
    mask = ids_ref[...] == cols
    gm = jnp.max(jnp.where(mask, g, -1e30), axis=0, keepdims=True)
    mn = jnp.sum(jnp.where(mask, gm, 0.0), axis=1, keepdims=True)
    e = jnp.exp(g - mn)
    dsum = jnp.sum(jnp.where(mask, e, 0.0), axis=0, keepdims=True)
    dn = jnp.sum(jnp.where(mask, dsum, 0.0), axis=1, keepdims=True)
    wgt = x * (e / dn)
    res = lax.dot_general(mask.astype(jnp.float32), wgt,
                          (((0,), (0,)), ((), ())),
                          preferred_element_type=jnp.float32)
    out_ref[...] = res[:_G, :]


_R = 2048


def _tc_dense1(x, W, a_s, a_d):
    return pl.pallas_call(
        _t1_body,
        grid=(_NP // _R,),
        in_specs=[
            pl.BlockSpec((_R, _H), lambda i: (i, 0)),
            pl.BlockSpec((_H, _H), lambda i: (0, 0)),
            pl.BlockSpec((_H, 1), lambda i: (0, 0)),
            pl.BlockSpec((_H, 1), lambda i: (0, 0)),
        ],
        out_specs=[
            pl.BlockSpec((_R, _H), lambda i: (i, 0)),
            pl.BlockSpec((_R, 1), lambda i: (i, 0)),
            pl.BlockSpec((_R, 1), lambda i: (i, 0)),
            pl.BlockSpec((1, 1), lambda i: (0, 0)),
        ],
        out_shape=[
            jax.ShapeDtypeStruct((_NP, _H), jnp.float32),
            jax.ShapeDtypeStruct((_NP, 1), jnp.float32),
            jax.ShapeDtypeStruct((_NP, 1), jnp.float32),
            jax.ShapeDtypeStruct((1, 1), jnp.float32),
        ],
        compiler_params=pltpu.CompilerParams(
            dimension_semantics=("arbitrary",)),
    )(x, W, a_s, a_d)


def _tc_dense2(acc, den, b, W, a_s, a_d):
    return pl.pallas_call(
        _t2_body,
        grid=(_NP // _R,),
        in_specs=[
            pl.BlockSpec((2, _R, _HC), lambda i: (0, i, 0)),
            pl.BlockSpec((2, _R, 1), lambda i: (0, i, 0)),
            pl.BlockSpec((1, _H), lambda i: (0, 0)),
            pl.BlockSpec((_H, _H), lambda i: (0, 0)),
            pl.BlockSpec((_H, 1), lambda i: (0, 0)),
            pl.BlockSpec((_H, 1), lambda i: (0, 0)),
        ],
        out_specs=[
            pl.BlockSpec((_R, _H), lambda i: (i, 0)),
            pl.BlockSpec((_R, 1), lambda i: (i, 0)),
            pl.BlockSpec((_R, 1), lambda i: (i, 0)),
            pl.BlockSpec((1, 1), lambda i: (0, 0)),
        ],
        out_shape=[
            jax.ShapeDtypeStruct((_NP, _H), jnp.float32),
            jax.ShapeDtypeStruct((_NP, 1), jnp.float32),
            jax.ShapeDtypeStruct((_NP, 1), jnp.float32),
            jax.ShapeDtypeStruct((1, 1), jnp.float32),
        ],
        compiler_params=pltpu.CompilerParams(
            dimension_semantics=("arbitrary",)),
    )(acc, den, b, W, a_s, a_d)


def _tc_final(acc, den, b, gw, gb, ids):
    return pl.pallas_call(
        _t3_body,
        out_shape=jax.ShapeDtypeStruct((_G, _H), jnp.float32),
    )(acc, den, b, gw, gb, ids)


def kernel(input_ids, attention_mask, edge_index, input_ids_batch, embed_table,
           W1, a_src1, a_dst1, b1, W2, a_src2, a_dst2, b2, gate_W, gate_b):
    f32 = jnp.float32
    last = input_ids[:, -1].astype(jnp.int32)
    idx3 = jnp.concatenate(
        [last, jnp.zeros((_NP - _N,), jnp.int32)]).reshape(_NW, _NE, _KE)
    x = _sc_embed_kernel()(embed_table.astype(f32), idx3)

    loop = jnp.arange(_N, dtype=jnp.int32)
    padn = _EP - _E - _N
    pad_src = jnp.arange(padn, dtype=jnp.int32) % 240
    pad_dst = _N + jnp.arange(padn, dtype=jnp.int32) % 240
    esrc = jnp.concatenate([edge_index[0].astype(jnp.int32), loop, pad_src])
    edst = jnp.concatenate([edge_index[1].astype(jnp.int32), loop, pad_dst])
    srcl = (2 * esrc).reshape(16, _NCH * _KC)
    srch = (2 * esrc + 1).reshape(16, _NCH * _KC)
    dst3 = edst.reshape(16, _NCH, _KC)

    def _layer(h, ssx, sdx, mgx):
        return _sc_edge_kernel()(
            srcl, srch, dst3, jnp.repeat(ssx.reshape(_NP), 2),
            sdx.reshape(_NP), jnp.broadcast_to(mgx[0, 0], (16,)),
            h.reshape(2 * _NP, _HC))

    h1, ss1, sd1, mg1 = _tc_dense1(
        x, W1, a_src1.reshape(_H, 1), a_dst1.reshape(_H, 1))
    acc1, den1 = _layer(h1, ss1, sd1, mg1)
    h2, ss2, sd2, mg2 = _tc_dense2(
        acc1, den1.reshape(2, _NP, 1), b1.reshape(1, _H),
        W2, a_src2.reshape(_H, 1), a_dst2.reshape(_H, 1))
    acc2, den2 = _layer(h2, ss2, sd2, mg2)
    out = _tc_final(acc2[:, :_N, :], den2[:, :_N].reshape(2, _N, 1),
                    b2.reshape(1, _H), gate_W, gate_b.reshape(1, 1),
                    input_ids_batch.astype(jnp.int32).reshape(_N, 1))
    return out


# speedup vs baseline: 1.0636x; 1.0263x over previous
import functools

import jax
import jax.numpy as jnp
from jax import lax
from jax.experimental import pallas as pl
from jax.experimental.pallas import tpu as pltpu
from jax.experimental.pallas import tpu_sc as plsc

_N = 10000
_H = 128
_G = 64
_NP = 10240
_E = 320000
_EP = 331776
_NW = 32
_HC = _H // 2
_KC = 128
_NCH = _EP // 16 // _KC
_RPT = _NP // 16
_BPW = _NP // _NW
_KE = 64
_NE = _BPW // _KE

@functools.cache
def _sc_embed_kernel():
    mesh = plsc.VectorSubcoreMesh(core_axis_name="c", subcore_axis_name="s")

    @functools.partial(
        pl.kernel, mesh=mesh,
        out_type=jax.ShapeDtypeStruct((_NP, _H), jnp.float32),
        scratch_types=[
            pltpu.VMEM((_NE, _KE), jnp.int32),
            [pltpu.VMEM((_KE, _H), jnp.float32)] * 2,
            [pltpu.SemaphoreType.DMA] * 2,
            [pltpu.SemaphoreType.DMA] * 2,
        ],
        compiler_params=pltpu.CompilerParams(needs_layout_passes=False),
    )
    def _sc_embed(tab, idx, x_out, idx_v, rows, gsem, ssem):
        w = lax.axis_index("s") * 2 + lax.axis_index("c")
        pltpu.sync_copy(idx.at[w], idx_v)

        def _out_at(q):
            return x_out.at[pl.ds(w * _BPW + q * _KE, _KE)]

        pltpu.async_copy(tab.at[idx_v.at[0]], rows[0], gsem[0])
        for q in range(_NE):
            b, nb = q % 2, (q + 1) % 2
            if q + 1 < _NE:
                if q >= 1:
                    pltpu.make_async_copy(rows[nb], _out_at(q - 1),
                                          ssem[nb]).wait()
                pltpu.async_copy(tab.at[idx_v.at[q + 1]], rows[nb], gsem[nb])
            pltpu.make_async_copy(tab.at[idx_v.at[q]], rows[b],
                                  gsem[b]).wait()
            pltpu.async_copy(rows[b], _out_at(q), ssem[b])
        for q in range(_NE - 2, _NE):
            pltpu.make_async_copy(rows[q % 2], _out_at(q),
                                  ssem[q % 2]).wait()

    return _sc_embed


@functools.cache
def _sc_edge_kernel():
    mesh = plsc.VectorSubcoreMesh(core_axis_name="c", subcore_axis_name="s")
    scratch_types = [
        pltpu.VMEM((_NCH * _KC,), jnp.int32),
        pltpu.VMEM((_NCH, _KC), jnp.int32),
        [pltpu.VMEM((_KC,), jnp.float32)] * 2,
        [pltpu.VMEM((_KC,), jnp.float32)] * 2,
        pltpu.VMEM((16,), jnp.float32),
        [pltpu.VMEM((_KC, _HC), jnp.float32)] * 3,
        [pltpu.VMEM((_KC,), jnp.float32)] * 2,
        pltpu.VMEM((_RPT,), jnp.float32),
        pltpu.VMEM_SHARED((_NP, _HC), jnp.float32),
        pltpu.VMEM_SHARED((_NP,), jnp.float32),
        [pltpu.SemaphoreType.DMA] * 3,
        [pltpu.SemaphoreType.DMA] * 2,
        [pltpu.SemaphoreType.DMA] * 2,
        [pltpu.SemaphoreType.DMA] * 3,
        [pltpu.SemaphoreType.DMA] * 2,
    ]

    @functools.partial(
        pl.kernel, mesh=mesh,
        out_type=[jax.ShapeDtypeStruct((2, _NP, _HC), jnp.float32),
                  jax.ShapeDtypeStruct((2, _NP), jnp.float32)],
        scratch_types=scratch_types,
        compiler_params=pltpu.CompilerParams(needs_layout_passes=False,
                                             use_tc_tiling_on_sc=False),
    )
    def _sc_edge(srcl, srch, dst, ss2, sd, mg, hil, acc_out, den_out,
                 src_v, dst_v, ssg, sdg, mg_v, rows, ex, zden_v,
                 acc_sh, den_sh, gsem, sssem, sdsem, scsem, dnsem):
        _sc_edge_body(srcl, srch, dst, ss2, sd, mg, hil, acc_out, den_out,
                      src_v, dst_v, ssg, sdg, mg_v, rows, ex, zden_v,
                      acc_sh, den_sh, gsem, sssem, sdsem, scsem, dnsem)

    return _sc_edge


def _sc_edge_body(srcl, srch, dst, ss2, sd, mg, hil, acc_out, den_out,
                  src_v, dst_v, ssg, sdg, mg_v, rows, ex, zden_v,
                  acc_sh, den_sh, gsem, sssem, sdsem, scsem, dnsem):
    c = lax.axis_index("c")
    s = lax.axis_index("s")
    pltpu.sync_copy(mg, mg_v)

    @pl.when(c == 0)
    def _():
        pltpu.sync_copy(srcl.at[s], src_v)

    @pl.when(c == 1)
    def _():
        pltpu.sync_copy(srch.at[s], src_v)
    pltpu.sync_copy(dst.at[s], dst_v)

    zv = jnp.zeros((16,), jnp.float32)

    def _zrow(j, carry):
        for cg in range(_HC // 16):
            rows[0][j, pl.ds(cg * 16, 16)] = zv
        return carry
    lax.fori_loop(0, _KC, _zrow, 0)

    def _zden(i, carry):
        zden_v[pl.ds(pl.multiple_of(i * 16, 16), 16)] = zv
        return carry
    lax.fori_loop(0, _RPT // 16, _zden, 0)

    base = s * _RPT
    for q in range(_RPT // _KC):
        pltpu.sync_copy(rows[0], acc_sh.at[pl.ds(base + q * _KC, _KC)])
    pltpu.sync_copy(zden_v, den_sh.at[pl.ds(base, _RPT)])
    plsc.subcore_barrier()

    mgv = mg_v[...]

    def _sidx(k):
        off = k * _KC if isinstance(k, int) else pl.multiple_of(k * _KC, 8)
        return src_v.at[pl.ds(off, _KC)]

    def _when(cond, fn):
        if isinstance(cond, bool):
            if cond:
                fn()
        else:
            pl.when(cond)(fn)

    def _issue(k, b3, b2):
        pltpu.async_copy(hil.at[_sidx(k)], rows[b3], gsem[b3])
        pltpu.async_copy(ss2.at[_sidx(k)], ssg[b2], sssem[b2])
        pltpu.async_copy(sd.at[dst_v.at[k]], sdg[b2], sdsem[b2])

    def _wait_scatter(k, b3):
        pltpu.make_async_copy(rows[b3], acc_sh.at[dst_v.at[k]],
                              scsem[b3]).wait()

    def _wait_den(k, b2):
        pltpu.make_async_copy(ex[b2], den_sh.at[dst_v.at[k]],
                              dnsem[b2]).wait()

    def _process(k, b3, b2):
        pltpu.make_async_copy(ss2.at[_sidx(k)], ssg[b2], sssem[b2]).wait()
        pltpu.make_async_copy(sd.at[dst_v.at[k]], sdg[b2], sdsem[b2]).wait()

        def _grp(g, carry2):
            off = pl.multiple_of(g * 16, 16)
            av = ssg[b2][pl.ds(off, 16)]
            bv = sdg[b2][pl.ds(off, 16)]
            lg = av + bv
            lg = jnp.maximum(lg, 0.2 * lg)
            mm = mgv + bv
            mm = jnp.maximum(mm, 0.2 * mm)
            ex[b2][pl.ds(off, 16)] = jnp.exp(lg - mm)
            return carry2
        lax.fori_loop(0, _KC // 16, _grp, 0)
        pltpu.async_copy(ex[b2], den_sh.at[dst_v.at[k]], dnsem[b2], add=True)

        pltpu.make_async_copy(hil.at[_sidx(k)], rows[b3], gsem[b3]).wait()

        def _scale(jj, carry2):
            for u in range(2):
                j = jj * 2 + u
                ej = plsc.load_gather(ex[b2],
                                      [jnp.zeros((16,), jnp.int32) + j])
                for cg in range(_HC // 16):
                    sl = pl.ds(cg * 16, 16)
                    rows[b3][j, sl] = rows[b3][j, sl] * ej
            return carry2
        lax.fori_loop(0, _KC // 2, _scale, 0)

        pltpu.async_copy(rows[b3], acc_sh.at[dst_v.at[k]], scsem[b3],
                         add=True)

    def _step(k, b3, b2, issue_next):
        if issue_next:
            _when(k >= 2, lambda: _wait_scatter(k - 2, (b3 + 1) % 3))
            _issue(k + 1, (b3 + 1) % 3, (b2 + 1) % 2)

        _when(k >= 2, lambda: _wait_den(k - 2, b2))

        _process(k, b3, b2)

    _issue(0, 0, 0)

    def _macro(jj, carry):
        for i in range(6):
            k = jj * 6 + i
            _step(k, i % 3, i % 2, True)
        return carry
    lax.fori_loop(0, _NCH // 6 - 1, _macro, 0)

    for i in range(6):
        k = _NCH - 6 + i
        _step(k, i % 3, i % 2, issue_next=(i < 5))
    for k in (_NCH - 3, _NCH - 2, _NCH - 1):
        _wait_scatter(k, k % 3)
    for k in (_NCH - 2, _NCH - 1):
        _wait_den(k, k % 2)
    plsc.subcore_barrier()

    pltpu.sync_copy(acc_sh.at[pl.ds(base, _RPT)],
                    acc_out.at[c, pl.ds(base, _RPT)])
    pltpu.sync_copy(den_sh.at[pl.ds(base, _RPT)],
                    den_out.at[c, pl.ds(base, _RPT)])


def _t1_body(x_ref, w_ref, as_ref, ad_ref, h_ref, ss_ref, sd_ref, mg_ref):
    h = jnp.dot(x_ref[...], w_ref[...], preferred_element_type=jnp.float32)
    h_ref[...] = h
    ss = jnp.dot(h, as_ref[...], preferred_element_type=jnp.float32)
    sd = jnp.dot(h, ad_ref[...], preferred_element_type=jnp.float32)
    ss_ref[...] = jnp.concatenate([ss, ss], axis=1)
    sd_ref[...] = sd

    @pl.when(pl.program_id(0) == 0)
    def _():
        mg_ref[...] = jnp.full((1, 16), -1e30, jnp.float32)
    mg_ref[...] = jnp.maximum(mg_ref[...], jnp.max(ss))


def _t2_body(acc_ref, den_ref, b_ref, w_ref, as_ref, ad_ref,
             h_ref, ss_ref, sd_ref, mg_ref):
    den = den_ref[0]
    den = jnp.where(den == 0.0, 1.0, den)
    acc = jnp.concatenate([acc_ref[0], acc_ref[1]], axis=1)
    x = jnp.maximum(acc / den + b_ref[...], 0.0)
    h = jnp.dot(x, w_ref[...], preferred_element_type=jnp.float32)
    h_ref[...] = h
    ss = jnp.dot(h, as_ref[...], preferred_element_type=jnp.float32)
    sd = jnp.dot(h, ad_ref[...], preferred_element_type=jnp.float32)
    ss_ref[...] = jnp.concatenate([ss, ss], axis=1)
    sd_ref[...] = sd

    @pl.when(pl.program_id(0) == 0)
    def _():
        mg_ref[...] = jnp.full((1, 16), -1e30, jnp.float32)
    mg_ref[...] = jnp.maximum(mg_ref[...], jnp.max(ss))


def _t3_body(acc_ref, den_ref, b_ref, gw_ref, gb_ref, ids_ref, out_ref):
    den = den_ref[0]
    den = jnp.where(den == 0.0, 1.0, den)
    acc = jnp.concatenate([acc_ref[0], acc_ref[1]], axis=1)
    x = jnp.maximum(acc / den + b_ref[...], 0.0)
    g = jnp.dot(x, gw_ref[...], preferred_element_type=jnp.float32) + gb_ref[...]
    cols = lax.broadcasted_iota(jnp.int32, (_N, _H), 1)
    mask = ids_ref[...] == cols
    gm = jnp.max(jnp.where(mask, g, -1e30), axis=0, keepdims=True)
    mn = jnp.sum(jnp.where(mask, gm, 0.0), axis=1, keepdims=True)
    e = jnp.exp(g - mn)
    dsum = jnp.sum(jnp.where(mask, e, 0.0), axis=0, keepdims=True)
    dn = jnp.sum(jnp.where(mask, dsum, 0.0), axis=1, keepdims=True)
    wgt = x * (e / dn)
    res = lax.dot_general(mask.astype(jnp.float32), wgt,
                          (((0,), (0,)), ((), ())),
                          preferred_element_type=jnp.float32)
    out_ref[...] = res[:_G, :]


_R = 2048


def _tc_dense1(x, W, a_s, a_d):
    return pl.pallas_call(
        _t1_body,
        grid=(_NP // _R,),
        in_specs=[
            pl.BlockSpec((_R, _H), lambda i: (i, 0)),
            pl.BlockSpec((_H, _H), lambda i: (0, 0)),
            pl.BlockSpec((_H, 1), lambda i: (0, 0)),
            pl.BlockSpec((_H, 1), lambda i: (0, 0)),
        ],
        out_specs=[
            pl.BlockSpec((_R, _H), lambda i: (i, 0)),
            pl.BlockSpec((_R, 2), lambda i: (i, 0)),
            pl.BlockSpec((_R, 1), lambda i: (i, 0)),
            pl.BlockSpec((1, 16), lambda i: (0, 0)),
        ],
        out_shape=[
            jax.ShapeDtypeStruct((_NP, _H), jnp.float32),
            jax.ShapeDtypeStruct((_NP, 2), jnp.float32),
            jax.ShapeDtypeStruct((_NP, 1), jnp.float32),
            jax.ShapeDtypeStruct((1, 16), jnp.float32),
        ],
        compiler_params=pltpu.CompilerParams(
            dimension_semantics=("arbitrary",)),
    )(x, W, a_s, a_d)


def _tc_dense2(acc, den, b, W, a_s, a_d):
    return pl.pallas_call(
        _t2_body,
        grid=(_NP // _R,),
        in_specs=[
            pl.BlockSpec((2, _R, _HC), lambda i: (0, i, 0)),
            pl.BlockSpec((2, _R, 1), lambda i: (0, i, 0)),
            pl.BlockSpec((1, _H), lambda i: (0, 0)),
            pl.BlockSpec((_H, _H), lambda i: (0, 0)),
            pl.BlockSpec((_H, 1), lambda i: (0, 0)),
            pl.BlockSpec((_H, 1), lambda i: (0, 0)),
        ],
        out_specs=[
            pl.BlockSpec((_R, _H), lambda i: (i, 0)),
            pl.BlockSpec((_R, 2), lambda i: (i, 0)),
            pl.BlockSpec((_R, 1), lambda i: (i, 0)),
            pl.BlockSpec((1, 16), lambda i: (0, 0)),
        ],
        out_shape=[
            jax.ShapeDtypeStruct((_NP, _H), jnp.float32),
            jax.ShapeDtypeStruct((_NP, 2), jnp.float32),
            jax.ShapeDtypeStruct((_NP, 1), jnp.float32),
            jax.ShapeDtypeStruct((1, 16), jnp.float32),
        ],
        compiler_params=pltpu.CompilerParams(
            dimension_semantics=("arbitrary",)),
    )(acc, den, b, W, a_s, a_d)


def _tc_final(acc, den, b, gw, gb, ids):
    return pl.pallas_call(
        _t3_body,
        out_shape=jax.ShapeDtypeStruct((_G, _H), jnp.float32),
    )(acc, den, b, gw, gb, ids)


def kernel(input_ids, attention_mask, edge_index, input_ids_batch, embed_table,
           W1, a_src1, a_dst1, b1, W2, a_src2, a_dst2, b2, gate_W, gate_b):
    f32 = jnp.float32
    last = input_ids[:, -1].astype(jnp.int32)
    idx3 = jnp.concatenate(
        [last, jnp.zeros((_NP - _N,), jnp.int32)]).reshape(_NW, _NE, _KE)
    x = _sc_embed_kernel()(embed_table.astype(f32), idx3)

    loop = jnp.arange(_N, dtype=jnp.int32)
    padn = _EP - _E - _N
    pad_src = jnp.arange(padn, dtype=jnp.int32) % 240
    pad_dst = _N + jnp.arange(padn, dtype=jnp.int32) % 240
    esrc = jnp.concatenate([edge_index[0].astype(jnp.int32), loop, pad_src])
    edst = jnp.concatenate([edge_index[1].astype(jnp.int32), loop, pad_dst])
    srcl = (2 * esrc).reshape(16, _NCH * _KC)
    srch = (2 * esrc + 1).reshape(16, _NCH * _KC)
    dst3 = edst.reshape(16, _NCH, _KC)

    def _layer(h, ssx, sdx, mgx):
        return _sc_edge_kernel()(
            srcl, srch, dst3, ssx.reshape(2 * _NP),
            sdx.reshape(_NP), mgx.reshape(16), h.reshape(2 * _NP, _HC))

    h1, ss1, sd1, mg1 = _tc_dense1(
        x, W1, a_src1.reshape(_H, 1), a_dst1.reshape(_H, 1))
    acc1, den1 = _layer(h1, ss1, sd1, mg1)
    h2, ss2, sd2, mg2 = _tc_dense2(
        acc1, den1.reshape(2, _NP, 1), b1.reshape(1, _H),
        W2, a_src2.reshape(_H, 1), a_dst2.reshape(_H, 1))
    acc2, den2 = _layer(h2, ss2, sd2, mg2)
    out = _tc_final(acc2[:, :_N, :], den2[:, :_N].reshape(2, _N, 1),
                    b2.reshape(1, _H), gate_W, gate_b.reshape(1, 1),
                    input_ids_batch.astype(jnp.int32).reshape(_N, 1))
    return out
